# baseline (speedup 1.0000x reference)
import functools

import jax
import jax.numpy as jnp
from jax import lax
from jax.experimental import pallas as pl
from jax.experimental.pallas import tpu as pltpu
from jax.experimental.pallas import tpu_sc as plsc

_N = 10000
_E = 320000
_G = 64
_EPS = 1e-7
_BN_EPS = 1e-5

_NCORE = 2
_NSUB = 16
_EPAD = 327680
_KC = 64
_CPW = 160
_CPW0 = 224
_CPW1 = 96
_NPAD = 10240
_RPS = _NPAD // _NSUB

_HI = jax.lax.Precision.HIGHEST


def _sc_edge_pass(x_tab, emat_slab, src2, dst64, C, off, mode, n0=_CPW, n1=_CPW):
    mesh = plsc.VectorSubcoreMesh(
        core_axis_name="c", subcore_axis_name="s",
        num_cores=_NCORE, num_subcores=_NSUB)
    wb = mode == "gather_wb"
    gather = mode != "linear"

    outs = jax.ShapeDtypeStruct((_NCORE, _NPAD, 128), jnp.float32)
    if wb:
        outs = (outs, jax.ShapeDtypeStruct((_EPAD, 128), jnp.float32))

    @functools.partial(
        pl.kernel,
        out_type=outs,
        mesh=mesh,
        scratch_types=[
            pltpu.VMEM_SHARED((_NPAD, 128), jnp.float32),
            pltpu.VMEM((8, 128), jnp.int32),
            pltpu.VMEM((8, 64), jnp.int32),
            pltpu.VMEM((_KC, 128), jnp.float32),
            pltpu.VMEM((_KC, 128), jnp.float32),
            pltpu.VMEM((_KC, C), jnp.float32),
            pltpu.VMEM((_KC, 128), jnp.float32),
            pltpu.VMEM((_KC, 128), jnp.float32),
            pltpu.SemaphoreType.DMA,
            pltpu.SemaphoreType.DMA,
            pltpu.SemaphoreType.DMA,
            pltpu.SemaphoreType.DMA,
            pltpu.SemaphoreType.DMA,
            pltpu.SemaphoreType.DMA,
            pltpu.SemaphoreType.DMA,
        ],
    )
    def k(*refs):
        if wb:
            (x_hbm, emat_hbm, src_hbm, dst_hbm, out_hbm, xsrc_hbm,
             acc, isrcb, idstb, xr0, xr1, erb, v0, v1,
             sg0, sg1, se0, ss0, ss1, sw0, sw1) = refs
        else:
            (x_hbm, emat_hbm, src_hbm, dst_hbm, out_hbm,
             acc, isrcb, idstb, xr0, xr1, erb, v0, v1,
             sg0, sg1, se0, ss0, ss1, sw0, sw1) = refs
        c = lax.axis_index("c")
        s = lax.axis_index("s")
        nc = jnp.where(c == 0, n0, n1)
        cbase = jnp.where(c == 0, s * n0, _NSUB * n0 + s * n1)
        xr = (xr0, xr1)
        vv = (v0, v1)
        sg = (sg0, sg1)
        ss = (ss0, ss1)
        sw = (sw0, sw1)

        def vzrow(i, carry):
            for j in range(8):
                v0[i, pl.ds(16 * j, 16)] = jnp.zeros((16,), jnp.float32)
                v1[i, pl.ds(16 * j, 16)] = jnp.zeros((16,), jnp.float32)
            return carry
        lax.fori_loop(0, _KC, vzrow, 0)

        base = s * _RPS
        for t in range(_RPS // _KC):
            pltpu.sync_copy(v0, acc.at[pl.ds(base + _KC * t, _KC)])
        plsc.subcore_barrier()

        def load_src_block(blk):
            pltpu.sync_copy(src_hbm.at[pl.ds(pl.multiple_of(cbase // 2 + blk * 8, 8), 8)],
                            isrcb)

        def load_dst_block(blk):
            pltpu.sync_copy(dst_hbm.at[pl.ds(pl.multiple_of(cbase + blk * 8, 8), 8)],
                            idstb)

        if gather:
            load_src_block(0)
        load_dst_block(0)

        wbase = cbase * _KC

        def issue_g(k_, b):
            if gather:
                rr = lax.rem(k_, 16) // 2
                hf = lax.rem(k_, 2)
                pltpu.async_copy(
                    x_hbm.at[isrcb.at[rr, pl.ds(hf * _KC, _KC)]], xr[b], sg[b])
            else:
                eb = pl.multiple_of(wbase + k_ * _KC, 64)
                pltpu.async_copy(x_hbm.at[pl.ds(eb, _KC)], xr[b], sg[b])

        def wait_g(b):
            pltpu.make_async_copy(x_hbm.at[pl.ds(0, _KC)], xr[b], sg[b]).wait()

        def issue_e(k_):
            eb = pl.multiple_of(wbase + k_ * _KC, 64)
            pltpu.async_copy(emat_hbm.at[pl.ds(eb, _KC)], erb, se0)

        def wait_e():
            pltpu.make_async_copy(emat_hbm.at[pl.ds(0, _KC)], erb, se0).wait()

        def issue_w(k_, b):
            if wb:
                eb = pl.multiple_of(wbase + k_ * _KC, 64)
                pltpu.async_copy(xr[b], xsrc_hbm.at[pl.ds(eb, _KC)], sw[b])

        def wait_w(b):
            if wb:
                pltpu.make_async_copy(xr[b], xsrc_hbm.at[pl.ds(0, _KC)],
                                      sw[b]).wait()

        def compute(b):
            v = vv[b]
            xb = xr[b]

            def edge(kk, ecarry):
                for j in range(C // 16):
                    xv = xb[kk, pl.ds(off + 16 * j, 16)]
                    ev = erb[kk, pl.ds(16 * j, 16)]
                    m = jnp.maximum(xv + ev, 0.0) + _EPS
                    wv = jnp.exp(m)
                    v[kk, pl.ds(16 * j, 16)] = wv * m
                    v[kk, pl.ds(C + 16 * j, 16)] = wv
                return ecarry
            lax.fori_loop(0, _KC, edge, 0)

        def issue_s(k_, b):
            pltpu.async_copy(vv[b], acc.at[idstb.at[lax.rem(k_, 8)]],
                             ss[b], add=True)

        def wait_s(b):
            pltpu.make_async_copy(vv[b], acc.at[idstb.at[0]], ss[b]).wait()

        issue_g(0, 0)
        issue_e(0)
        wait_g(0)
        issue_w(0, 0)
        issue_g(1, 1)
        wait_e()
        compute(0)
        issue_e(1)
        issue_s(0, 0)

        def steady(t, carry):
            for (k_, b) in ((2 * t + 1, 1), (2 * t + 2, 0)):
                wait_s(1 - b)

                @pl.when((lax.rem(k_, 8) == 0) & (k_ > 0))
                def _():
                    load_dst_block(k_ // 8)
                wait_g(b)
                issue_w(k_, b)
                if gather:
                    @pl.when(lax.rem(k_ + 1, 16) == 0)
                    def _():
                        load_src_block((k_ + 1) // 16)
                wait_w(1 - b)
                issue_g(k_ + 1, 1 - b)
                wait_e()
                compute(b)
                issue_e(k_ + 1)
                issue_s(k_, b)
            return carry
        lax.fori_loop(0, (nc - 2) // 2, steady, 0)

        wait_s(0)
        wait_g(1)
        issue_w(nc - 1, 1)
        wait_e()
        compute(1)
        issue_s(nc - 1, 1)
        wait_s(1)
        wait_w(0)
        wait_w(1)
        plsc.subcore_barrier()

        pltpu.sync_copy(acc.at[pl.ds(base, _RPS)],
                        out_hbm.at[c, pl.ds(base, _RPS)])

    if wb:
        return k(x_tab, emat_slab, src2, dst64)
    return k(x_tab, emat_slab, src2, dst64)


def _tc_edge_matmul(edge_attr, We, be, slabs):
    cin = We.shape[1]
    BE = 4096
    grid = (_EPAD // BE,)

    def kern(ea_ref, we_ref, be_ref, *out_refs):
        e = jnp.dot(ea_ref[...], we_ref[...], precision=_HI,
                    preferred_element_type=jnp.float32) + be_ref[...]
        off = 0
        for r, cs in zip(out_refs, slabs):
            r[...] = e[:, off:off + cs]
            off += cs

    return pl.pallas_call(
        kern,
        grid=grid,
        in_specs=[pl.BlockSpec((BE, 16), lambda i: (i, 0)),
                  pl.BlockSpec((16, cin), lambda i: (0, 0)),
                  pl.BlockSpec((1, cin), lambda i: (0, 0))],
        out_specs=[pl.BlockSpec((BE, cs), lambda i: (i, 0)) for cs in slabs],
        out_shape=[jax.ShapeDtypeStruct((_EPAD, cs), jnp.float32) for cs in slabs],
    )(edge_attr, We, be.reshape(1, cin))


def _tc_combine_w1(parts, slabs, x, W1, b1):
    cin = W1.shape[0]
    c2 = W1.shape[1]
    RB = 1000
    grid = (_N // RB,)
    npart = len(parts)

    def kern(*refs):
        part_refs = refs[:npart]
        x_ref, w1_ref, b1_ref, h1_ref, st_ref = refs[npart:]
        i = pl.program_id(0)
        aggs = []
        for r, cs in zip(part_refs, slabs):
            num = r[0, :, :cs] + r[1, :, :cs]
            den = r[0, :, cs:2 * cs] + r[1, :, cs:2 * cs]
            aggs.append(num / (den + 1e-16))
        agg = jnp.concatenate(aggs, axis=1) if npart > 1 else aggs[0]
        h = x_ref[:, :cin] + agg
        h1 = jnp.dot(h, w1_ref[...], precision=_HI,
                     preferred_element_type=jnp.float32) + b1_ref[...]
        h1_ref[...] = h1

        @pl.when(i == 0)
        def _():
            st_ref[...] = jnp.zeros_like(st_ref)
        st_ref[...] += jnp.concatenate(
            [jnp.sum(h1, axis=0, keepdims=True),
             jnp.sum(h1 * h1, axis=0, keepdims=True)], axis=0)

    return pl.pallas_call(
        kern,
        grid=grid,
        in_specs=(
            [pl.BlockSpec((2, RB, 128), lambda i: (0, i, 0)) for _ in slabs]
            + [pl.BlockSpec((RB, x.shape[1]), lambda i: (i, 0)),
               pl.BlockSpec((cin, c2), lambda i: (0, 0)),
               pl.BlockSpec((1, c2), lambda i: (0, 0))]),
        out_specs=[pl.BlockSpec((RB, c2), lambda i: (i, 0)),
                   pl.BlockSpec((2, c2), lambda i: (0, 0))],
        out_shape=[jax.ShapeDtypeStruct((_N, c2), jnp.float32),
                   jax.ShapeDtypeStruct((2, c2), jnp.float32)],
    )(*parts, x, W1, b1.reshape(1, c2))


def _tc_bn_relu_w2(h1, st1, g1, bn1, W2, b2):
    c2 = h1.shape[1]
    cout = W2.shape[1]
    RB = 1000
    grid = (_N // RB,)

    def kern(h1_ref, st_ref, g_ref, b_ref, w2_ref, b2_ref, h2_ref, st2_ref):
        i = pl.program_id(0)
        mu = st_ref[0:1, :] * (1.0 / _N)
        var = st_ref[1:2, :] * (1.0 / _N) - mu * mu
        t = (h1_ref[...] - mu) * lax.rsqrt(var + _BN_EPS) * g_ref[...] + b_ref[...]
        t = jnp.maximum(t, 0.0)
        h2 = jnp.dot(t, w2_ref[...], precision=_HI,
                     preferred_element_type=jnp.float32) + b2_ref[...]
        h2_ref[...] = h2

        @pl.when(i == 0)
        def _():
            st2_ref[...] = jnp.zeros_like(st2_ref)
        st2_ref[...] += jnp.concatenate(
            [jnp.sum(h2, axis=0, keepdims=True),
             jnp.sum(h2 * h2, axis=0, keepdims=True)], axis=0)

    return pl.pallas_call(
        kern,
        grid=grid,
        in_specs=[pl.BlockSpec((RB, c2), lambda i: (i, 0)),
                  pl.BlockSpec((2, c2), lambda i: (0, 0)),
                  pl.BlockSpec((1, c2), lambda i: (0, 0)),
                  pl.BlockSpec((1, c2), lambda i: (0, 0)),
                  pl.BlockSpec((c2, cout), lambda i: (0, 0)),
                  pl.BlockSpec((1, cout), lambda i: (0, 0))],
        out_specs=[pl.BlockSpec((RB, cout), lambda i: (i, 0)),
                   pl.BlockSpec((2, cout), lambda i: (0, 0))],
        out_shape=[jax.ShapeDtypeStruct((_N, cout), jnp.float32),
                   jax.ShapeDtypeStruct((2, cout), jnp.float32)],
    )(h1, st1, g1.reshape(1, c2), bn1.reshape(1, c2), W2, b2.reshape(1, cout))


def _tc_bn_leaky(h2, st2, g, b):
    cout = h2.shape[1]
    RB = 1000
    grid = (_N // RB,)

    def kern(h2_ref, st_ref, g_ref, b_ref, o_ref):
        mu = st_ref[0:1, :] * (1.0 / _N)
        var = st_ref[1:2, :] * (1.0 / _N) - mu * mu
        t = (h2_ref[...] - mu) * lax.rsqrt(var + _BN_EPS) * g_ref[...] + b_ref[...]
        t = jnp.where(t >= 0, t, 0.01 * t)
        if cout < 128:
            t = jnp.concatenate(
                [t, jnp.zeros((RB, 128 - cout), jnp.float32)], axis=1)
        o_ref[...] = t

    return pl.pallas_call(
        kern,
        grid=grid,
        in_specs=[pl.BlockSpec((RB, cout), lambda i: (i, 0)),
                  pl.BlockSpec((2, cout), lambda i: (0, 0)),
                  pl.BlockSpec((1, cout), lambda i: (0, 0)),
                  pl.BlockSpec((1, cout), lambda i: (0, 0))],
        out_specs=pl.BlockSpec((RB, 128), lambda i: (i, 0)),
        out_shape=jax.ShapeDtypeStruct((_N, 128), jnp.float32),
    )(h2, st2, g.reshape(1, cout), b.reshape(1, cout))


def _tc_pool(h, batch3):
    cout = h.shape[1]
    RB = 1000
    grid = (_N // RB,)

    def kern(h_ref, b_ref, o_ref, cnt_ref):
        i = pl.program_id(0)

        @pl.when(i == 0)
        def _():
            o_ref[...] = jnp.zeros_like(o_ref)
            cnt_ref[...] = jnp.zeros_like(cnt_ref)
        bids = b_ref[0, 0, :]
        oh = (bids[None, :] ==
              lax.broadcasted_iota(jnp.int32, (_G, RB), 0)).astype(jnp.float32)
        o_ref[...] += jnp.dot(oh, h_ref[...], precision=_HI,
                              preferred_element_type=jnp.float32)
        cnt_ref[...] += jnp.broadcast_to(
            jnp.sum(oh, axis=1, keepdims=True), (_G, cout))

        @pl.when(i == grid[0] - 1)
        def _():
            o_ref[...] = o_ref[...] / jnp.maximum(cnt_ref[...], 1.0)

    return pl.pallas_call(
        kern,
        grid=grid,
        in_specs=[pl.BlockSpec((RB, cout), lambda i: (i, 0)),
                  pl.BlockSpec((1, 1, RB), lambda i: (i, 0, 0))],
        out_specs=pl.BlockSpec((_G, cout), lambda i: (0, 0)),
        out_shape=jax.ShapeDtypeStruct((_G, cout), jnp.float32),
        scratch_shapes=[pltpu.VMEM((_G, cout), jnp.float32)],
    )(h, batch3)


def _layer(h, edge_attr, src, dst, p, norm_g, norm_b, slabs):
    cin = p["W1"].shape[0]
    emats = _tc_edge_matmul(edge_attr, p["We"], p["be"], [cs for cs, _ in slabs])
    parts = []
    xsrc = None
    for i, (emat_s, (cs, off)) in enumerate(zip(emats, slabs)):
        if len(slabs) > 1 and i == 0:
            part, xsrc = _sc_edge_pass(h, emat_s, src, dst, cs, off,
                                       "gather_wb", _CPW0, _CPW1)
        elif xsrc is not None:
            part = _sc_edge_pass(xsrc, emat_s, src, dst, cs, off, "linear")
        else:
            part = _sc_edge_pass(h, emat_s, src, dst, cs, off, "gather",
                                 _CPW0, _CPW1)
        parts.append(part)
    h1, st1 = _tc_combine_w1(parts, [cs for cs, _ in slabs], h, p["W1"], p["b1"])
    h2, st2 = _tc_bn_relu_w2(h1, st1, p["g1"], p["bn1"], p["W2"], p["b2"])
    return _tc_bn_leaky(h2, st2, norm_g, norm_b)


def kernel(x, edge_attr, params, edge_index, batch):
    src = edge_index[0]
    dst = edge_index[1]
    pad = _EPAD - _E
    ea_pad = jnp.concatenate(
        [edge_attr, jnp.zeros((pad, edge_attr.shape[1]), jnp.float32)], axis=0)
    src2 = jnp.concatenate(
        [src, jnp.zeros((pad,), src.dtype)], axis=0).reshape(_EPAD // 128, 128)
    dummy_dst = (_N + jnp.arange(pad, dtype=dst.dtype) % (_NPAD - _N))
    dst2 = jnp.concatenate([dst, dummy_dst], axis=0).reshape(_EPAD // 64, 64)
    batch3 = batch.reshape(_N // 1000, 1, 1000)
    h = _layer(x, ea_pad, src2, dst2, params["conv1"],
               params["norm1_g"], params["norm1_b"], ((64, 0), (64, 64)))
    h = _layer(h, ea_pad, src2, dst2, params["conv2"],
               params["norm2_g"], params["norm2_b"], ((32, 0),))
    h = _layer(h, ea_pad, src2, dst2, params["conv3"],
               params["norm3_g"], params["norm3_b"], ((64, 0),))
    return _tc_pool(h[:, :128], batch3)


# speedup vs baseline: 1.0198x; 1.0198x over previous
import functools

import jax
import jax.numpy as jnp
from jax import lax
from jax.experimental import pallas as pl
from jax.experimental.pallas import tpu as pltpu
from jax.experimental.pallas import tpu_sc as plsc

_N = 10000
_E = 320000
_G = 64
_EPS = 1e-7
_BN_EPS = 1e-5

_NCORE = 2
_NSUB = 16
_EPAD = 327680
_KC = 64
_CPW = 160
_CPW0 = 256
_CPW1 = 64
_NPAD = 10240
_RPS = _NPAD // _NSUB

_HI = jax.lax.Precision.HIGHEST


def _sc_edge_pass(x_tab, emat_slab, src2, dst64, C, off, mode, n0=_CPW, n1=_CPW):
    mesh = plsc.VectorSubcoreMesh(
        core_axis_name="c", subcore_axis_name="s",
        num_cores=_NCORE, num_subcores=_NSUB)
    wb = mode == "gather_wb"
    gather = mode != "linear"

    outs = jax.ShapeDtypeStruct((_NCORE, _NPAD, 128), jnp.float32)
    if wb:
        outs = (outs, jax.ShapeDtypeStruct((_EPAD, 128), jnp.float32))

    @functools.partial(
        pl.kernel,
        out_type=outs,
        mesh=mesh,
        scratch_types=[
            pltpu.VMEM_SHARED((_NPAD, 128), jnp.float32),
            pltpu.VMEM((8, 128), jnp.int32),
            pltpu.VMEM((8, 64), jnp.int32),
            pltpu.VMEM((_KC, 128), jnp.float32),
            pltpu.VMEM((_KC, 128), jnp.float32),
            pltpu.VMEM((_KC, C), jnp.float32),
            pltpu.VMEM((_KC, 128), jnp.float32),
            pltpu.VMEM((_KC, 128), jnp.float32),
            pltpu.SemaphoreType.DMA,
            pltpu.SemaphoreType.DMA,
            pltpu.SemaphoreType.DMA,
            pltpu.SemaphoreType.DMA,
            pltpu.SemaphoreType.DMA,
            pltpu.SemaphoreType.DMA,
            pltpu.SemaphoreType.DMA,
        ],
    )
    def k(*refs):
        if wb:
            (x_hbm, emat_hbm, src_hbm, dst_hbm, out_hbm, xsrc_hbm,
             acc, isrcb, idstb, xr0, xr1, erb, v0, v1,
             sg0, sg1, se0, ss0, ss1, sw0, sw1) = refs
        else:
            (x_hbm, emat_hbm, src_hbm, dst_hbm, out_hbm,
             acc, isrcb, idstb, xr0, xr1, erb, v0, v1,
             sg0, sg1, se0, ss0, ss1, sw0, sw1) = refs
        c = lax.axis_index("c")
        s = lax.axis_index("s")
        nc = jnp.where(c == 0, n0, n1)
        cbase = jnp.where(c == 0, s * n0, _NSUB * n0 + s * n1)
        xr = (xr0, xr1)
        vv = (v0, v1)
        sg = (sg0, sg1)
        ss = (ss0, ss1)
        sw = (sw0, sw1)

        def vzrow(i, carry):
            for j in range(8):
                v0[i, pl.ds(16 * j, 16)] = jnp.zeros((16,), jnp.float32)
                v1[i, pl.ds(16 * j, 16)] = jnp.zeros((16,), jnp.float32)
            return carry
        lax.fori_loop(0, _KC, vzrow, 0)

        base = s * _RPS
        for t in range(_RPS // _KC):
            pltpu.sync_copy(v0, acc.at[pl.ds(base + _KC * t, _KC)])
        plsc.subcore_barrier()

        def load_src_block(blk):
            pltpu.sync_copy(src_hbm.at[pl.ds(pl.multiple_of(cbase // 2 + blk * 8, 8), 8)],
                            isrcb)

        def load_dst_block(blk):
            pltpu.sync_copy(dst_hbm.at[pl.ds(pl.multiple_of(cbase + blk * 8, 8), 8)],
                            idstb)

        if gather:
            load_src_block(0)
        load_dst_block(0)

        wbase = cbase * _KC

        def issue_g(k_, b):
            if gather:
                rr = lax.rem(k_, 16) // 2
                hf = lax.rem(k_, 2)
                pltpu.async_copy(
                    x_hbm.at[isrcb.at[rr, pl.ds(hf * _KC, _KC)]], xr[b], sg[b])
            else:
                eb = pl.multiple_of(wbase + k_ * _KC, 64)
                pltpu.async_copy(x_hbm.at[pl.ds(eb, _KC)], xr[b], sg[b])

        def wait_g(b):
            pltpu.make_async_copy(x_hbm.at[pl.ds(0, _KC)], xr[b], sg[b]).wait()

        def issue_e(k_):
            eb = pl.multiple_of(wbase + k_ * _KC, 64)
            pltpu.async_copy(emat_hbm.at[pl.ds(eb, _KC)], erb, se0)

        def wait_e():
            pltpu.make_async_copy(emat_hbm.at[pl.ds(0, _KC)], erb, se0).wait()

        def issue_w(k_, b):
            if wb:
                eb = pl.multiple_of(wbase + k_ * _KC, 64)
                pltpu.async_copy(xr[b], xsrc_hbm.at[pl.ds(eb, _KC)], sw[b])

        def wait_w(b):
            if wb:
                pltpu.make_async_copy(xr[b], xsrc_hbm.at[pl.ds(0, _KC)],
                                      sw[b]).wait()

        def compute(b):
            v = vv[b]
            xb = xr[b]

            def edge(kk, ecarry):
                for j in range(C // 16):
                    xv = xb[kk, pl.ds(off + 16 * j, 16)]
                    ev = erb[kk, pl.ds(16 * j, 16)]
                    m = jnp.maximum(xv + ev, 0.0) + _EPS
                    wv = jnp.exp(m)
                    v[kk, pl.ds(16 * j, 16)] = wv * m
                    v[kk, pl.ds(C + 16 * j, 16)] = wv
                return ecarry
            lax.fori_loop(0, _KC, edge, 0)

        def issue_s(k_, b):
            pltpu.async_copy(vv[b], acc.at[idstb.at[lax.rem(k_, 8)]],
                             ss[b], add=True)

        def wait_s(b):
            pltpu.make_async_copy(vv[b], acc.at[idstb.at[0]], ss[b]).wait()

        issue_g(0, 0)
        issue_e(0)
        wait_g(0)
        issue_w(0, 0)
        issue_g(1, 1)
        wait_e()
        compute(0)
        issue_e(1)
        issue_s(0, 0)

        def steady(t, carry):
            for (k_, b) in ((2 * t + 1, 1), (2 * t + 2, 0)):
                wait_s(1 - b)

                @pl.when((lax.rem(k_, 8) == 0) & (k_ > 0))
                def _():
                    load_dst_block(k_ // 8)
                wait_g(b)
                issue_w(k_, b)
                if gather:
                    @pl.when(lax.rem(k_ + 1, 16) == 0)
                    def _():
                        load_src_block((k_ + 1) // 16)
                wait_w(1 - b)
                issue_g(k_ + 1, 1 - b)
                wait_e()
                compute(b)
                issue_e(k_ + 1)
                issue_s(k_, b)
            return carry
        lax.fori_loop(0, (nc - 2) // 2, steady, 0)

        wait_s(0)
        wait_g(1)
        issue_w(nc - 1, 1)
        wait_e()
        compute(1)
        issue_s(nc - 1, 1)
        wait_s(1)
        wait_w(0)
        wait_w(1)
        plsc.subcore_barrier()

        pltpu.sync_copy(acc.at[pl.ds(base, _RPS)],
                        out_hbm.at[c, pl.ds(base, _RPS)])

    if wb:
        return k(x_tab, emat_slab, src2, dst64)
    return k(x_tab, emat_slab, src2, dst64)


def _tc_edge_matmul(edge_attr, We, be, slabs):
    cin = We.shape[1]
    BE = 4096
    grid = (_EPAD // BE,)

    def kern(ea_ref, we_ref, be_ref, *out_refs):
        e = jnp.dot(ea_ref[...], we_ref[...], precision=_HI,
                    preferred_element_type=jnp.float32) + be_ref[...]
        off = 0
        for r, cs in zip(out_refs, slabs):
            r[...] = e[:, off:off + cs]
            off += cs

    return pl.pallas_call(
        kern,
        grid=grid,
        in_specs=[pl.BlockSpec((BE, 16), lambda i: (i, 0)),
                  pl.BlockSpec((16, cin), lambda i: (0, 0)),
                  pl.BlockSpec((1, cin), lambda i: (0, 0))],
        out_specs=[pl.BlockSpec((BE, cs), lambda i: (i, 0)) for cs in slabs],
        out_shape=[jax.ShapeDtypeStruct((_EPAD, cs), jnp.float32) for cs in slabs],
    )(edge_attr, We, be.reshape(1, cin))


def _tc_combine_w1(parts, slabs, x, W1, b1):
    cin = W1.shape[0]
    c2 = W1.shape[1]
    RB = 1000
    grid = (_N // RB,)
    npart = len(parts)

    def kern(*refs):
        part_refs = refs[:npart]
        x_ref, w1_ref, b1_ref, h1_ref, st_ref = refs[npart:]
        i = pl.program_id(0)
        aggs = []
        for r, cs in zip(part_refs, slabs):
            num = r[0, :, :cs] + r[1, :, :cs]
            den = r[0, :, cs:2 * cs] + r[1, :, cs:2 * cs]
            aggs.append(num / (den + 1e-16))
        agg = jnp.concatenate(aggs, axis=1) if npart > 1 else aggs[0]
        h = x_ref[:, :cin] + agg
        h1 = jnp.dot(h, w1_ref[...], precision=_HI,
                     preferred_element_type=jnp.float32) + b1_ref[...]
        h1_ref[...] = h1

        @pl.when(i == 0)
        def _():
            st_ref[...] = jnp.zeros_like(st_ref)
        st_ref[...] += jnp.concatenate(
            [jnp.sum(h1, axis=0, keepdims=True),
             jnp.sum(h1 * h1, axis=0, keepdims=True)], axis=0)

    return pl.pallas_call(
        kern,
        grid=grid,
        in_specs=(
            [pl.BlockSpec((2, RB, 128), lambda i: (0, i, 0)) for _ in slabs]
            + [pl.BlockSpec((RB, x.shape[1]), lambda i: (i, 0)),
               pl.BlockSpec((cin, c2), lambda i: (0, 0)),
               pl.BlockSpec((1, c2), lambda i: (0, 0))]),
        out_specs=[pl.BlockSpec((RB, c2), lambda i: (i, 0)),
                   pl.BlockSpec((2, c2), lambda i: (0, 0))],
        out_shape=[jax.ShapeDtypeStruct((_N, c2), jnp.float32),
                   jax.ShapeDtypeStruct((2, c2), jnp.float32)],
    )(*parts, x, W1, b1.reshape(1, c2))


def _tc_bn_relu_w2(h1, st1, g1, bn1, W2, b2):
    c2 = h1.shape[1]
    cout = W2.shape[1]
    RB = 1000
    grid = (_N // RB,)

    def kern(h1_ref, st_ref, g_ref, b_ref, w2_ref, b2_ref, h2_ref, st2_ref):
        i = pl.program_id(0)
        mu = st_ref[0:1, :] * (1.0 / _N)
        var = st_ref[1:2, :] * (1.0 / _N) - mu * mu
        t = (h1_ref[...] - mu) * lax.rsqrt(var + _BN_EPS) * g_ref[...] + b_ref[...]
        t = jnp.maximum(t, 0.0)
        h2 = jnp.dot(t, w2_ref[...], precision=_HI,
                     preferred_element_type=jnp.float32) + b2_ref[...]
        h2_ref[...] = h2

        @pl.when(i == 0)
        def _():
            st2_ref[...] = jnp.zeros_like(st2_ref)
        st2_ref[...] += jnp.concatenate(
            [jnp.sum(h2, axis=0, keepdims=True),
             jnp.sum(h2 * h2, axis=0, keepdims=True)], axis=0)

    return pl.pallas_call(
        kern,
        grid=grid,
        in_specs=[pl.BlockSpec((RB, c2), lambda i: (i, 0)),
                  pl.BlockSpec((2, c2), lambda i: (0, 0)),
                  pl.BlockSpec((1, c2), lambda i: (0, 0)),
                  pl.BlockSpec((1, c2), lambda i: (0, 0)),
                  pl.BlockSpec((c2, cout), lambda i: (0, 0)),
                  pl.BlockSpec((1, cout), lambda i: (0, 0))],
        out_specs=[pl.BlockSpec((RB, cout), lambda i: (i, 0)),
                   pl.BlockSpec((2, cout), lambda i: (0, 0))],
        out_shape=[jax.ShapeDtypeStruct((_N, cout), jnp.float32),
                   jax.ShapeDtypeStruct((2, cout), jnp.float32)],
    )(h1, st1, g1.reshape(1, c2), bn1.reshape(1, c2), W2, b2.reshape(1, cout))


def _tc_bn_leaky(h2, st2, g, b):
    cout = h2.shape[1]
    RB = 1000
    grid = (_N // RB,)

    def kern(h2_ref, st_ref, g_ref, b_ref, o_ref):
        mu = st_ref[0:1, :] * (1.0 / _N)
        var = st_ref[1:2, :] * (1.0 / _N) - mu * mu
        t = (h2_ref[...] - mu) * lax.rsqrt(var + _BN_EPS) * g_ref[...] + b_ref[...]
        t = jnp.where(t >= 0, t, 0.01 * t)
        if cout < 128:
            t = jnp.concatenate(
                [t, jnp.zeros((RB, 128 - cout), jnp.float32)], axis=1)
        o_ref[...] = t

    return pl.pallas_call(
        kern,
        grid=grid,
        in_specs=[pl.BlockSpec((RB, cout), lambda i: (i, 0)),
                  pl.BlockSpec((2, cout), lambda i: (0, 0)),
                  pl.BlockSpec((1, cout), lambda i: (0, 0)),
                  pl.BlockSpec((1, cout), lambda i: (0, 0))],
        out_specs=pl.BlockSpec((RB, 128), lambda i: (i, 0)),
        out_shape=jax.ShapeDtypeStruct((_N, 128), jnp.float32),
    )(h2, st2, g.reshape(1, cout), b.reshape(1, cout))


def _tc_pool(h, batch3):
    cout = h.shape[1]
    RB = 1000
    grid = (_N // RB,)

    def kern(h_ref, b_ref, o_ref, cnt_ref):
        i = pl.program_id(0)

        @pl.when(i == 0)
        def _():
            o_ref[...] = jnp.zeros_like(o_ref)
            cnt_ref[...] = jnp.zeros_like(cnt_ref)
        bids = b_ref[0, 0, :]
        oh = (bids[None, :] ==
              lax.broadcasted_iota(jnp.int32, (_G, RB), 0)).astype(jnp.float32)
        o_ref[...] += jnp.dot(oh, h_ref[...], precision=_HI,
                              preferred_element_type=jnp.float32)
        cnt_ref[...] += jnp.broadcast_to(
            jnp.sum(oh, axis=1, keepdims=True), (_G, cout))

        @pl.when(i == grid[0] - 1)
        def _():
            o_ref[...] = o_ref[...] / jnp.maximum(cnt_ref[...], 1.0)

    return pl.pallas_call(
        kern,
        grid=grid,
        in_specs=[pl.BlockSpec((RB, cout), lambda i: (i, 0)),
                  pl.BlockSpec((1, 1, RB), lambda i: (i, 0, 0))],
        out_specs=pl.BlockSpec((_G, cout), lambda i: (0, 0)),
        out_shape=jax.ShapeDtypeStruct((_G, cout), jnp.float32),
        scratch_shapes=[pltpu.VMEM((_G, cout), jnp.float32)],
    )(h, batch3)


def _layer(h, edge_attr, src, dst, p, norm_g, norm_b, slabs):
    cin = p["W1"].shape[0]
    emats = _tc_edge_matmul(edge_attr, p["We"], p["be"], [cs for cs, _ in slabs])
    parts = []
    xsrc = None
    for i, (emat_s, (cs, off)) in enumerate(zip(emats, slabs)):
        if len(slabs) > 1 and i == 0:
            part, xsrc = _sc_edge_pass(h, emat_s, src, dst, cs, off,
                                       "gather_wb", _CPW0, _CPW1)
        elif xsrc is not None:
            part = _sc_edge_pass(xsrc, emat_s, src, dst, cs, off, "linear")
        else:
            part = _sc_edge_pass(h, emat_s, src, dst, cs, off, "gather",
                                 _CPW0, _CPW1)
        parts.append(part)
    h1, st1 = _tc_combine_w1(parts, [cs for cs, _ in slabs], h, p["W1"], p["b1"])
    h2, st2 = _tc_bn_relu_w2(h1, st1, p["g1"], p["bn1"], p["W2"], p["b2"])
    return _tc_bn_leaky(h2, st2, norm_g, norm_b)


def kernel(x, edge_attr, params, edge_index, batch):
    src = edge_index[0]
    dst = edge_index[1]
    pad = _EPAD - _E
    ea_pad = jnp.concatenate(
        [edge_attr, jnp.zeros((pad, edge_attr.shape[1]), jnp.float32)], axis=0)
    src2 = jnp.concatenate(
        [src, jnp.zeros((pad,), src.dtype)], axis=0).reshape(_EPAD // 128, 128)
    dummy_dst = (_N + jnp.arange(pad, dtype=dst.dtype) % (_NPAD - _N))
    dst2 = jnp.concatenate([dst, dummy_dst], axis=0).reshape(_EPAD // 64, 64)
    batch3 = batch.reshape(_N // 1000, 1, 1000)
    h = _layer(x, ea_pad, src2, dst2, params["conv1"],
               params["norm1_g"], params["norm1_b"], ((64, 0), (64, 64)))
    h = _layer(h, ea_pad, src2, dst2, params["conv2"],
               params["norm2_g"], params["norm2_b"], ((32, 0),))
    h = _layer(h, ea_pad, src2, dst2, params["conv3"],
               params["norm3_g"], params["norm3_b"], ((64, 0),))
    return _tc_pool(h[:, :128], batch3)


# speedup vs baseline: 1.0233x; 1.0034x over previous
import functools

import jax
import jax.numpy as jnp
from jax import lax
from jax.experimental import pallas as pl
from jax.experimental.pallas import tpu as pltpu
from jax.experimental.pallas import tpu_sc as plsc

_N = 10000
_E = 320000
_G = 64
_EPS = 1e-7
_BN_EPS = 1e-5

_NCORE = 2
_NSUB = 16
_EPAD = 327680
_KC = 64
_CPW = 160
_CPW0 = 272
_CPW1 = 48
_NPAD = 10240
_RPS = _NPAD // _NSUB

_HI = jax.lax.Precision.HIGHEST


def _sc_edge_pass(x_tab, emat_slab, src2, dst64, C, off, mode, n0=_CPW, n1=_CPW):
    mesh = plsc.VectorSubcoreMesh(
        core_axis_name="c", subcore_axis_name="s",
        num_cores=_NCORE, num_subcores=_NSUB)
    wb = mode == "gather_wb"
    gather = mode != "linear"

    outs = jax.ShapeDtypeStruct((_NCORE, _NPAD, 128), jnp.float32)
    if wb:
        outs = (outs, jax.ShapeDtypeStruct((_EPAD, 128), jnp.float32))

    @functools.partial(
        pl.kernel,
        out_type=outs,
        mesh=mesh,
        scratch_types=[
            pltpu.VMEM_SHARED((_NPAD, 128), jnp.float32),
            pltpu.VMEM((8, 128), jnp.int32),
            pltpu.VMEM((8, 64), jnp.int32),
            pltpu.VMEM((_KC, 128), jnp.float32),
            pltpu.VMEM((_KC, 128), jnp.float32),
            pltpu.VMEM((_KC, C), jnp.float32),
            pltpu.VMEM((_KC, 128), jnp.float32),
            pltpu.VMEM((_KC, 128), jnp.float32),
            pltpu.SemaphoreType.DMA,
            pltpu.SemaphoreType.DMA,
            pltpu.SemaphoreType.DMA,
            pltpu.SemaphoreType.DMA,
            pltpu.SemaphoreType.DMA,
            pltpu.SemaphoreType.DMA,
            pltpu.SemaphoreType.DMA,
        ],
    )
    def k(*refs):
        if wb:
            (x_hbm, emat_hbm, src_hbm, dst_hbm, out_hbm, xsrc_hbm,
             acc, isrcb, idstb, xr0, xr1, erb, v0, v1,
             sg0, sg1, se0, ss0, ss1, sw0, sw1) = refs
        else:
            (x_hbm, emat_hbm, src_hbm, dst_hbm, out_hbm,
             acc, isrcb, idstb, xr0, xr1, erb, v0, v1,
             sg0, sg1, se0, ss0, ss1, sw0, sw1) = refs
        c = lax.axis_index("c")
        s = lax.axis_index("s")
        nc = jnp.where(c == 0, n0, n1)
        cbase = jnp.where(c == 0, s * n0, _NSUB * n0 + s * n1)
        xr = (xr0, xr1)
        vv = (v0, v1)
        sg = (sg0, sg1)
        ss = (ss0, ss1)
        sw = (sw0, sw1)

        def vzrow(i, carry):
            for j in range(8):
                v0[i, pl.ds(16 * j, 16)] = jnp.zeros((16,), jnp.float32)
                v1[i, pl.ds(16 * j, 16)] = jnp.zeros((16,), jnp.float32)
            return carry
        lax.fori_loop(0, _KC, vzrow, 0)

        base = s * _RPS
        for t in range(_RPS // _KC):
            pltpu.sync_copy(v0, acc.at[pl.ds(base + _KC * t, _KC)])
        plsc.subcore_barrier()

        def load_src_block(blk):
            pltpu.sync_copy(src_hbm.at[pl.ds(pl.multiple_of(cbase // 2 + blk * 8, 8), 8)],
                            isrcb)

        def load_dst_block(blk):
            pltpu.sync_copy(dst_hbm.at[pl.ds(pl.multiple_of(cbase + blk * 8, 8), 8)],
                            idstb)

        if gather:
            load_src_block(0)
        load_dst_block(0)

        wbase = cbase * _KC

        def issue_g(k_, b):
            if gather:
                rr = lax.rem(k_, 16) // 2
                hf = lax.rem(k_, 2)
                pltpu.async_copy(
                    x_hbm.at[isrcb.at[rr, pl.ds(hf * _KC, _KC)]], xr[b], sg[b])
            else:
                eb = pl.multiple_of(wbase + k_ * _KC, 64)
                pltpu.async_copy(x_hbm.at[pl.ds(eb, _KC)], xr[b], sg[b])

        def wait_g(b):
            pltpu.make_async_copy(x_hbm.at[pl.ds(0, _KC)], xr[b], sg[b]).wait()

        def issue_e(k_):
            eb = pl.multiple_of(wbase + k_ * _KC, 64)
            pltpu.async_copy(emat_hbm.at[pl.ds(eb, _KC)], erb, se0)

        def wait_e():
            pltpu.make_async_copy(emat_hbm.at[pl.ds(0, _KC)], erb, se0).wait()

        def issue_w(k_, b):
            if wb:
                eb = pl.multiple_of(wbase + k_ * _KC, 64)
                pltpu.async_copy(xr[b], xsrc_hbm.at[pl.ds(eb, _KC)], sw[b])

        def wait_w(b):
            if wb:
                pltpu.make_async_copy(xr[b], xsrc_hbm.at[pl.ds(0, _KC)],
                                      sw[b]).wait()

        def compute(b):
            v = vv[b]
            xb = xr[b]

            def edge(kk, ecarry):
                for j in range(C // 16):
                    xv = xb[kk, pl.ds(off + 16 * j, 16)]
                    ev = erb[kk, pl.ds(16 * j, 16)]
                    m = jnp.maximum(xv + ev, 0.0) + _EPS
                    wv = jnp.exp(m)
                    v[kk, pl.ds(16 * j, 16)] = wv * m
                    v[kk, pl.ds(C + 16 * j, 16)] = wv
                return ecarry
            lax.fori_loop(0, _KC, edge, 0)

        def issue_s(k_, b):
            pltpu.async_copy(vv[b], acc.at[idstb.at[lax.rem(k_, 8)]],
                             ss[b], add=True)

        def wait_s(b):
            pltpu.make_async_copy(vv[b], acc.at[idstb.at[0]], ss[b]).wait()

        issue_g(0, 0)
        issue_e(0)
        wait_g(0)
        issue_w(0, 0)
        issue_g(1, 1)
        wait_e()
        compute(0)
        issue_e(1)
        issue_s(0, 0)

        def steady(t, carry):
            for (k_, b) in ((2 * t + 1, 1), (2 * t + 2, 0)):
                wait_s(1 - b)

                @pl.when((lax.rem(k_, 8) == 0) & (k_ > 0))
                def _():
                    load_dst_block(k_ // 8)
                wait_g(b)
                issue_w(k_, b)
                if gather:
                    @pl.when(lax.rem(k_ + 1, 16) == 0)
                    def _():
                        load_src_block((k_ + 1) // 16)
                wait_w(1 - b)
                issue_g(k_ + 1, 1 - b)
                wait_e()
                compute(b)
                issue_e(k_ + 1)
                issue_s(k_, b)
            return carry
        lax.fori_loop(0, (nc - 2) // 2, steady, 0)

        wait_s(0)
        wait_g(1)
        issue_w(nc - 1, 1)
        wait_e()
        compute(1)
        issue_s(nc - 1, 1)
        wait_s(1)
        wait_w(0)
        wait_w(1)
        plsc.subcore_barrier()

        pltpu.sync_copy(acc.at[pl.ds(base, _RPS)],
                        out_hbm.at[c, pl.ds(base, _RPS)])

    if wb:
        return k(x_tab, emat_slab, src2, dst64)
    return k(x_tab, emat_slab, src2, dst64)


def _tc_edge_matmul(edge_attr, We, be, slabs):
    cin = We.shape[1]
    BE = 4096
    grid = (_EPAD // BE,)

    def kern(ea_ref, we_ref, be_ref, *out_refs):
        e = jnp.dot(ea_ref[...], we_ref[...], precision=_HI,
                    preferred_element_type=jnp.float32) + be_ref[...]
        off = 0
        for r, cs in zip(out_refs, slabs):
            r[...] = e[:, off:off + cs]
            off += cs

    return pl.pallas_call(
        kern,
        grid=grid,
        in_specs=[pl.BlockSpec((BE, 16), lambda i: (i, 0)),
                  pl.BlockSpec((16, cin), lambda i: (0, 0)),
                  pl.BlockSpec((1, cin), lambda i: (0, 0))],
        out_specs=[pl.BlockSpec((BE, cs), lambda i: (i, 0)) for cs in slabs],
        out_shape=[jax.ShapeDtypeStruct((_EPAD, cs), jnp.float32) for cs in slabs],
    )(edge_attr, We, be.reshape(1, cin))


def _tc_combine_w1(parts, slabs, x, W1, b1):
    cin = W1.shape[0]
    c2 = W1.shape[1]
    RB = 1000
    grid = (_N // RB,)
    npart = len(parts)

    def kern(*refs):
        part_refs = refs[:npart]
        x_ref, w1_ref, b1_ref, h1_ref, st_ref = refs[npart:]
        i = pl.program_id(0)
        aggs = []
        for r, cs in zip(part_refs, slabs):
            num = r[0, :, :cs] + r[1, :, :cs]
            den = r[0, :, cs:2 * cs] + r[1, :, cs:2 * cs]
            aggs.append(num / (den + 1e-16))
        agg = jnp.concatenate(aggs, axis=1) if npart > 1 else aggs[0]
        h = x_ref[:, :cin] + agg
        h1 = jnp.dot(h, w1_ref[...], precision=_HI,
                     preferred_element_type=jnp.float32) + b1_ref[...]
        h1_ref[...] = h1

        @pl.when(i == 0)
        def _():
            st_ref[...] = jnp.zeros_like(st_ref)
        st_ref[...] += jnp.concatenate(
            [jnp.sum(h1, axis=0, keepdims=True),
             jnp.sum(h1 * h1, axis=0, keepdims=True)], axis=0)

    return pl.pallas_call(
        kern,
        grid=grid,
        in_specs=(
            [pl.BlockSpec((2, RB, 128), lambda i: (0, i, 0)) for _ in slabs]
            + [pl.BlockSpec((RB, x.shape[1]), lambda i: (i, 0)),
               pl.BlockSpec((cin, c2), lambda i: (0, 0)),
               pl.BlockSpec((1, c2), lambda i: (0, 0))]),
        out_specs=[pl.BlockSpec((RB, c2), lambda i: (i, 0)),
                   pl.BlockSpec((2, c2), lambda i: (0, 0))],
        out_shape=[jax.ShapeDtypeStruct((_N, c2), jnp.float32),
                   jax.ShapeDtypeStruct((2, c2), jnp.float32)],
    )(*parts, x, W1, b1.reshape(1, c2))


def _tc_bn_relu_w2(h1, st1, g1, bn1, W2, b2):
    c2 = h1.shape[1]
    cout = W2.shape[1]
    RB = 1000
    grid = (_N // RB,)

    def kern(h1_ref, st_ref, g_ref, b_ref, w2_ref, b2_ref, h2_ref, st2_ref):
        i = pl.program_id(0)
        mu = st_ref[0:1, :] * (1.0 / _N)
        var = st_ref[1:2, :] * (1.0 / _N) - mu * mu
        t = (h1_ref[...] - mu) * lax.rsqrt(var + _BN_EPS) * g_ref[...] + b_ref[...]
        t = jnp.maximum(t, 0.0)
        h2 = jnp.dot(t, w2_ref[...], precision=_HI,
                     preferred_element_type=jnp.float32) + b2_ref[...]
        h2_ref[...] = h2

        @pl.when(i == 0)
        def _():
            st2_ref[...] = jnp.zeros_like(st2_ref)
        st2_ref[...] += jnp.concatenate(
            [jnp.sum(h2, axis=0, keepdims=True),
             jnp.sum(h2 * h2, axis=0, keepdims=True)], axis=0)

    return pl.pallas_call(
        kern,
        grid=grid,
        in_specs=[pl.BlockSpec((RB, c2), lambda i: (i, 0)),
                  pl.BlockSpec((2, c2), lambda i: (0, 0)),
                  pl.BlockSpec((1, c2), lambda i: (0, 0)),
                  pl.BlockSpec((1, c2), lambda i: (0, 0)),
                  pl.BlockSpec((c2, cout), lambda i: (0, 0)),
                  pl.BlockSpec((1, cout), lambda i: (0, 0))],
        out_specs=[pl.BlockSpec((RB, cout), lambda i: (i, 0)),
                   pl.BlockSpec((2, cout), lambda i: (0, 0))],
        out_shape=[jax.ShapeDtypeStruct((_N, cout), jnp.float32),
                   jax.ShapeDtypeStruct((2, cout), jnp.float32)],
    )(h1, st1, g1.reshape(1, c2), bn1.reshape(1, c2), W2, b2.reshape(1, cout))


def _tc_bn_leaky(h2, st2, g, b):
    cout = h2.shape[1]
    RB = 1000
    grid = (_N // RB,)

    def kern(h2_ref, st_ref, g_ref, b_ref, o_ref):
        mu = st_ref[0:1, :] * (1.0 / _N)
        var = st_ref[1:2, :] * (1.0 / _N) - mu * mu
        t = (h2_ref[...] - mu) * lax.rsqrt(var + _BN_EPS) * g_ref[...] + b_ref[...]
        t = jnp.where(t >= 0, t, 0.01 * t)
        if cout < 128:
            t = jnp.concatenate(
                [t, jnp.zeros((RB, 128 - cout), jnp.float32)], axis=1)
        o_ref[...] = t

    return pl.pallas_call(
        kern,
        grid=grid,
        in_specs=[pl.BlockSpec((RB, cout), lambda i: (i, 0)),
                  pl.BlockSpec((2, cout), lambda i: (0, 0)),
                  pl.BlockSpec((1, cout), lambda i: (0, 0)),
                  pl.BlockSpec((1, cout), lambda i: (0, 0))],
        out_specs=pl.BlockSpec((RB, 128), lambda i: (i, 0)),
        out_shape=jax.ShapeDtypeStruct((_N, 128), jnp.float32),
    )(h2, st2, g.reshape(1, cout), b.reshape(1, cout))


def _tc_pool(h, batch3):
    cout = h.shape[1]
    RB = 1000
    grid = (_N // RB,)

    def kern(h_ref, b_ref, o_ref, cnt_ref):
        i = pl.program_id(0)

        @pl.when(i == 0)
        def _():
            o_ref[...] = jnp.zeros_like(o_ref)
            cnt_ref[...] = jnp.zeros_like(cnt_ref)
        bids = b_ref[0, 0, :]
        oh = (bids[None, :] ==
              lax.broadcasted_iota(jnp.int32, (_G, RB), 0)).astype(jnp.float32)
        o_ref[...] += jnp.dot(oh, h_ref[...], precision=_HI,
                              preferred_element_type=jnp.float32)
        cnt_ref[...] += jnp.broadcast_to(
            jnp.sum(oh, axis=1, keepdims=True), (_G, cout))

        @pl.when(i == grid[0] - 1)
        def _():
            o_ref[...] = o_ref[...] / jnp.maximum(cnt_ref[...], 1.0)

    return pl.pallas_call(
        kern,
        grid=grid,
        in_specs=[pl.BlockSpec((RB, cout), lambda i: (i, 0)),
                  pl.BlockSpec((1, 1, RB), lambda i: (i, 0, 0))],
        out_specs=pl.BlockSpec((_G, cout), lambda i: (0, 0)),
        out_shape=jax.ShapeDtypeStruct((_G, cout), jnp.float32),
        scratch_shapes=[pltpu.VMEM((_G, cout), jnp.float32)],
    )(h, batch3)


def _layer(h, edge_attr, src, dst, p, norm_g, norm_b, slabs):
    cin = p["W1"].shape[0]
    emats = _tc_edge_matmul(edge_attr, p["We"], p["be"], [cs for cs, _ in slabs])
    parts = []
    xsrc = None
    for i, (emat_s, (cs, off)) in enumerate(zip(emats, slabs)):
        if len(slabs) > 1 and i == 0:
            part, xsrc = _sc_edge_pass(h, emat_s, src, dst, cs, off,
                                       "gather_wb", _CPW0, _CPW1)
        elif xsrc is not None:
            part = _sc_edge_pass(xsrc, emat_s, src, dst, cs, off, "linear")
        else:
            part = _sc_edge_pass(h, emat_s, src, dst, cs, off, "gather",
                                 _CPW0, _CPW1)
        parts.append(part)
    h1, st1 = _tc_combine_w1(parts, [cs for cs, _ in slabs], h, p["W1"], p["b1"])
    h2, st2 = _tc_bn_relu_w2(h1, st1, p["g1"], p["bn1"], p["W2"], p["b2"])
    return _tc_bn_leaky(h2, st2, norm_g, norm_b)


def kernel(x, edge_attr, params, edge_index, batch):
    src = edge_index[0]
    dst = edge_index[1]
    pad = _EPAD - _E
    ea_pad = jnp.concatenate(
        [edge_attr, jnp.zeros((pad, edge_attr.shape[1]), jnp.float32)], axis=0)
    src2 = jnp.concatenate(
        [src, jnp.zeros((pad,), src.dtype)], axis=0).reshape(_EPAD // 128, 128)
    dummy_dst = (_N + jnp.arange(pad, dtype=dst.dtype) % (_NPAD - _N))
    dst2 = jnp.concatenate([dst, dummy_dst], axis=0).reshape(_EPAD // 64, 64)
    batch3 = batch.reshape(_N // 1000, 1, 1000)
    h = _layer(x, ea_pad, src2, dst2, params["conv1"],
               params["norm1_g"], params["norm1_b"], ((64, 0), (64, 64)))
    h = _layer(h, ea_pad, src2, dst2, params["conv2"],
               params["norm2_g"], params["norm2_b"], ((32, 0),))
    h = _layer(h, ea_pad, src2, dst2, params["conv3"],
               params["norm3_g"], params["norm3_b"], ((64, 0),))
    return _tc_pool(h[:, :128], batch3)
